# baseline (speedup 1.0000x reference)
import jax
import jax.numpy as jnp
from jax import lax
from jax.experimental import pallas as pl
from jax.experimental.pallas import tpu as pltpu
from jax.experimental.pallas import tpu_sc as plsc

NBINS = 256
H = 512
W = 512
C = 3
NPIX = H * W
NTOT = H * W * C
NC = 2
NS = 16
NW = NC * NS
LANES = 16
CHUNK = NTOT // NW
HCHUNK = NTOT // NS
NHIST = 2 * C * NBINS
TCOLS = C * NBINS
SENTINEL = -3.0e38
G511 = 0.99609375
VB = NBINS // LANES


def _lane_iota():
    return lax.broadcasted_iota(jnp.int32, (LANES,), 0)


def _bin_index(v):
    vc = jnp.minimum(jnp.maximum(v, -1.0), 1.0)
    t = (vc + 1.0) * 128.0
    return jnp.minimum(t.astype(jnp.int32), NBINS - 1)


def _cdf_into(hbuf, off, cdfb, csbuf):
    lane = _lane_iota()
    zeros16 = lane * 0

    def cbody(k, carry):
        v = hbuf[pl.ds(off + k * LANES, LANES)]
        s = plsc.cumsum(v) + carry
        cdfb[pl.ds(k * LANES, LANES)] = s
        return zeros16 + jnp.max(s)

    lax.fori_loop(0, VB, cbody, zeros16)
    cdfmin = jnp.min(cdfb[pl.ds(0, LANES)])

    def sbody(k, _):
        d = cdfb[pl.ds(k * LANES, LANES)] - cdfmin
        csbuf[pl.ds(k * LANES, LANES)] = (
            d.astype(jnp.float32) * 2.0 / float(NPIX - 1) - 1.0)
        return _

    lax.fori_loop(0, VB, sbody, None)


def _mid_first(tab, mbuf, moff, fbuf):
    lane = _lane_iota()
    zeros16 = lane * 0

    def body(k, fcarry):
        jvec = lane + k * LANES
        cur = plsc.load_gather(tab, [jvec])
        prev = plsc.load_gather(tab, [jnp.maximum(jvec - 1, 0)])
        m = (prev + cur) * 0.5
        m = jnp.where(jvec == 0, SENTINEL, m)
        mbuf[pl.ds(moff + k * LANES, LANES)] = m
        cand = jnp.where(cur != prev, jvec, 0)
        f = jnp.maximum(plsc.cummax(cand), fcarry)
        fbuf[pl.ds(k * LANES, LANES)] = f
        return zeros16 + jnp.max(f)

    lax.fori_loop(0, VB, body, zeros16)


def _search(mbuf, base, x):
    r = base
    for step in (128, 64, 32, 16, 8, 4, 2, 1):
        probe = r + step
        mv = plsc.load_gather(mbuf, [probe])
        r = jnp.where(mv < x, probe, r)
    return r


def _fused_body(src_hbm, tgt_hbm, out_hbm, xbuf, priv, red, shared, tbuf,
                cdfb, csbuf, ctbuf, mtbuf, pmbuf, fbuf):
    sid = lax.axis_index("s")
    cid = lax.axis_index("c")
    wid = sid * NC + cid

    lane = _lane_iota()
    zeros16 = lane * 0
    ones16 = zeros16 + 1
    hoff = [lane * NHIST + lax.rem(lane + p, 3) * NBINS for p in range(3)]

    def zero_body(i, _):
        priv[pl.ds(i * LANES, LANES)] = zeros16
        return _

    lax.fori_loop(0, (LANES * NHIST) // LANES, zero_body, None)

    base_h = sid * HCHUNK
    for img, inp in ((0, src_hbm), (1, tgt_hbm)):
        pltpu.sync_copy(inp.at[pl.ds(base_h, HCHUNK)], xbuf)
        offs = [hoff[p] + img * C * NBINS for p in range(3)]

        def gbody(g, _, offs=offs):
            s0 = g * (3 * LANES)
            vs = [xbuf[pl.ds(s0 + p * LANES, LANES)] for p in range(3)]
            for p in range(3):
                idx = _bin_index(vs[p]) + offs[p]
                plsc.addupdate_scatter(priv, [idx], ones16)
            return _

        lax.fori_loop(0, HCHUNK // (3 * LANES), gbody, None)

    def rbody(k, _):
        acc = priv[pl.ds(k * LANES, LANES)]
        for l in range(1, LANES):
            acc = acc + priv[pl.ds(l * NHIST + k * LANES, LANES)]
        red[pl.ds(k * LANES, LANES)] = acc
        return _

    lax.fori_loop(0, NHIST // LANES, rbody, None)

    pltpu.sync_copy(red, shared.at[pl.ds(sid * NHIST, NHIST)])
    plsc.subcore_barrier()
    pltpu.sync_copy(shared, priv)

    def r2body(k, _):
        acc = priv[pl.ds(k * LANES, LANES)]
        for t in range(1, NS):
            acc = acc + priv[pl.ds(t * NHIST + k * LANES, LANES)]
        priv[pl.ds(k * LANES, LANES)] = acc
        return _

    lax.fori_loop(0, NHIST // LANES, r2body, None)

    inv256 = 1.0 / 256.0
    for c in range(C):
        _cdf_into(priv, c * NBINS, cdfb, csbuf)
        _cdf_into(priv, C * NBINS + c * NBINS, cdfb, ctbuf)

        _mid_first(ctbuf, mtbuf, 0, fbuf)

        def ibody(q, _):
            xs = csbuf[pl.ds(q * LANES, LANES)]
            ct255 = plsc.load_gather(ctbuf, [lane * 0 + (NBINS - 1)])
            r = _search(mtbuf, lane * 0, xs)
            ind1 = plsc.load_gather(fbuf, [r])
            ind0 = ind1 - 1
            neg = ind0 < 0
            i0_256 = jnp.where(neg, ind0 + NBINS, ind0)
            dx0 = plsc.load_gather(ctbuf, [i0_256])
            dx1 = plsc.load_gather(ctbuf, [ind1])
            dy0 = jnp.where(neg, ind0 + 2 * NBINS,
                            ind0).astype(jnp.float32) * inv256 - 1.0
            dy1 = ind1.astype(jnp.float32) * inv256 - 1.0
            interp = dy0 + (dy1 - dy0) * (xs - dx0) / (dx1 - dx0)
            pm = jnp.where(xs <= -1.0, -1.0,
                           jnp.where(xs >= ct255, G511, interp))
            pmbuf[pl.ds(q * LANES, LANES)] = pm
            return _

        lax.fori_loop(0, VB, ibody, None)

        _mid_first(csbuf, tbuf, c * NBINS, fbuf)

        def abody(k, _):
            rvec = lane + k * LANES
            i1 = plsc.load_gather(fbuf, [rvec])
            i0 = i1 - 1
            i0w = jnp.where(i0 < 0, i0 + NBINS, i0)
            dx0 = plsc.load_gather(csbuf, [i0w])
            dx1 = plsc.load_gather(csbuf, [i1])
            dy0 = plsc.load_gather(pmbuf, [i0w])
            dy1 = plsc.load_gather(pmbuf, [i1])
            b = (dy1 - dy0) / (dx1 - dx0)
            a = dy0 - b * dx0
            tbuf[pl.ds(TCOLS + c * NBINS + k * LANES, LANES)] = a
            tbuf[pl.ds(2 * TCOLS + c * NBINS + k * LANES, LANES)] = b
            return _

        lax.fori_loop(0, VB, abody, None)

        def clbody(k, _):
            cs255 = plsc.load_gather(csbuf, [lane * 0 + (NBINS - 1)])
            pm255 = plsc.load_gather(pmbuf, [lane * 0 + (NBINS - 1)])
            tbuf[pl.ds(3 * TCOLS + c * NBINS + k * LANES, LANES)] = cs255
            tbuf[pl.ds(4 * TCOLS + c * NBINS + k * LANES, LANES)] = pm255
            return _

        lax.fori_loop(0, VB, clbody, None)

    base_m = wid * CHUNK
    pltpu.sync_copy(src_hbm.at[pl.ds(base_m, CHUNK)],
                    xbuf.at[pl.ds(0, CHUNK)])
    ch256 = [lax.rem(lane + p, 3) * NBINS for p in range(3)]

    def mbody(g, _):
        s0 = g * (3 * LANES)
        xs = [xbuf[pl.ds(s0 + p * LANES, LANES)] for p in range(3)]
        rs = [ch256[p] for p in range(3)]
        for step in (128, 64, 32, 16, 8, 4, 2, 1):
            for p in range(3):
                probe = rs[p] + step
                mv = plsc.load_gather(tbuf, [probe])
                rs[p] = jnp.where(mv < xs[p], probe, rs[p])
        ys = []
        for p in range(3):
            a = plsc.load_gather(tbuf, [rs[p] + TCOLS])
            b = plsc.load_gather(tbuf, [rs[p] + 2 * TCOLS])
            thi = plsc.load_gather(tbuf, [rs[p] + 3 * TCOLS])
            vhi = plsc.load_gather(tbuf, [rs[p] + 4 * TCOLS])
            y = a + b * xs[p]
            y = jnp.where(xs[p] >= thi, vhi, y)
            ys.append(jnp.where(xs[p] <= -1.0, -1.0, y))
        for p in range(3):
            xbuf[pl.ds(s0 + p * LANES, LANES)] = ys[p]
        return _

    lax.fori_loop(0, CHUNK // (3 * LANES), mbody, None)
    pltpu.sync_copy(xbuf.at[pl.ds(0, CHUNK)],
                    out_hbm.at[pl.ds(base_m, CHUNK)])


def _sc_fused(src_f, tgt_f):
    mesh = plsc.VectorSubcoreMesh(
        core_axis_name="c", subcore_axis_name="s", num_cores=NC,
        num_subcores=NS)
    return pl.kernel(
        _fused_body,
        out_type=jax.ShapeDtypeStruct((NTOT,), jnp.float32),
        mesh=mesh,
        compiler_params=pltpu.CompilerParams(needs_layout_passes=False),
        scratch_types=[
            pltpu.VMEM((HCHUNK,), jnp.float32),
            pltpu.VMEM((LANES * NHIST,), jnp.int32),
            pltpu.VMEM((NHIST,), jnp.int32),
            pltpu.VMEM_SHARED((NS * NHIST,), jnp.int32),
            pltpu.VMEM((5 * TCOLS,), jnp.float32),
            pltpu.VMEM((NBINS,), jnp.int32),
            pltpu.VMEM((NBINS,), jnp.float32),
            pltpu.VMEM((NBINS,), jnp.float32),
            pltpu.VMEM((NBINS,), jnp.float32),
            pltpu.VMEM((NBINS,), jnp.float32),
            pltpu.VMEM((NBINS,), jnp.int32),
        ],
    )(src_f, tgt_f)


def kernel(src, tgt):
    src_f = src.reshape(-1)
    tgt_f = tgt.reshape(-1)
    out_f = _sc_fused(src_f, tgt_f)
    return out_f.reshape(H, W, C)


# speedup vs baseline: 3.1828x; 3.1828x over previous
import jax
import jax.numpy as jnp
from jax import lax
from jax.experimental import pallas as pl
from jax.experimental.pallas import tpu as pltpu
from jax.experimental.pallas import tpu_sc as plsc

NBINS = 256
H = 512
W = 512
C = 3
NPIX = H * W
NTOT = H * W * C
NC = 2
NS = 16
NW = NC * NS
LANES = 16
HSL = NPIX // NS
MSL = NPIX // NW
NHIST = 2 * C * NBINS
TCOLS = C * NBINS
SENTINEL = -3.0e38
G511 = 0.99609375
VB = NBINS // LANES
MU = 4


def _fwd(s):
    t = s.transpose(2, 0, 1).reshape(C, 64, 8, 4, 128)
    return t.transpose(0, 1, 3, 2, 4).reshape(-1)


def _bwd(f):
    t = f.reshape(C, 64, 4, 8, 128).transpose(0, 1, 3, 2, 4)
    return t.reshape(C, H, W).transpose(1, 2, 0)


def _lane_iota():
    return lax.broadcasted_iota(jnp.int32, (LANES,), 0)


def _bin_index(v):
    vc = jnp.minimum(jnp.maximum(v, -1.0), 1.0)
    t = (vc + 1.0) * 128.0
    return jnp.minimum(t.astype(jnp.int32), NBINS - 1)


def _cdf_into(hbuf, off, cdfb, csbuf):
    lane = _lane_iota()
    zeros16 = lane * 0

    def cbody(k, carry):
        v = hbuf[pl.ds(off + k * LANES, LANES)]
        s = plsc.cumsum(v) + carry
        cdfb[pl.ds(k * LANES, LANES)] = s
        return zeros16 + jnp.max(s)

    lax.fori_loop(0, VB, cbody, zeros16)
    cdfmin = jnp.min(cdfb[pl.ds(0, LANES)])

    def sbody(k, _):
        d = cdfb[pl.ds(k * LANES, LANES)] - cdfmin
        csbuf[pl.ds(k * LANES, LANES)] = (
            d.astype(jnp.float32) * 2.0 / float(NPIX - 1) - 1.0)
        return _

    lax.fori_loop(0, VB, sbody, None)


def _mid_first(tab, mbuf, moff, fbuf):
    lane = _lane_iota()
    zeros16 = lane * 0

    def body(k, fcarry):
        jvec = lane + k * LANES
        cur = plsc.load_gather(tab, [jvec])
        prev = plsc.load_gather(tab, [jnp.maximum(jvec - 1, 0)])
        m = (prev + cur) * 0.5
        m = jnp.where(jvec == 0, SENTINEL, m)
        mbuf[pl.ds(moff + k * LANES, LANES)] = m
        cand = jnp.where(cur != prev, jvec, 0)
        f = jnp.maximum(plsc.cummax(cand), fcarry)
        fbuf[pl.ds(k * LANES, LANES)] = f
        return zeros16 + jnp.max(f)

    lax.fori_loop(0, VB, body, zeros16)


def _search(mbuf, base, x):
    r = base
    for step in (128, 64, 32, 16, 8, 4, 2, 1):
        probe = r + step
        mv = plsc.load_gather(mbuf, [probe])
        r = jnp.where(mv < x, probe, r)
    return r


def _fused_body(src_hbm, tgt_hbm, out_hbm, xbuf, priv, red, shared, tbuf,
                cdfb, csbuf, ctbuf, mtbuf, pmbuf, fbuf):
    sid = lax.axis_index("s")
    cid = lax.axis_index("c")
    wid = sid * NC + cid

    lane = _lane_iota()
    zeros16 = lane * 0
    ones16 = zeros16 + 1

    def zero_body(i, _):
        priv[pl.ds(i * LANES, LANES)] = zeros16
        return _

    lax.fori_loop(0, (LANES * NHIST) // LANES, zero_body, None)

    for img, inp in ((0, src_hbm), (1, tgt_hbm)):
        for c in range(C):
            pltpu.sync_copy(inp.at[pl.ds(c * NPIX + sid * HSL, HSL)],
                            xbuf.at[pl.ds(0, HSL)])
            off = lane * NHIST + (img * C + c) * NBINS

            def hbody(g, _, off=off):
                for u in range(2):
                    v = xbuf[pl.ds((g * 2 + u) * LANES, LANES)]
                    idx = _bin_index(v) + off
                    plsc.addupdate_scatter(priv, [idx], ones16)
                return _

            lax.fori_loop(0, HSL // (2 * LANES), hbody, None)

    def rbody(k, _):
        acc = priv[pl.ds(k * LANES, LANES)]
        for l in range(1, LANES):
            acc = acc + priv[pl.ds(l * NHIST + k * LANES, LANES)]
        red[pl.ds(k * LANES, LANES)] = acc
        return _

    lax.fori_loop(0, NHIST // LANES, rbody, None)

    pltpu.sync_copy(red, shared.at[pl.ds(sid * NHIST, NHIST)])
    plsc.subcore_barrier()
    pltpu.sync_copy(shared, priv)

    def r2body(k, _):
        acc = priv[pl.ds(k * LANES, LANES)]
        for t in range(1, NS):
            acc = acc + priv[pl.ds(t * NHIST + k * LANES, LANES)]
        priv[pl.ds(k * LANES, LANES)] = acc
        return _

    lax.fori_loop(0, NHIST // LANES, r2body, None)

    inv256 = 1.0 / 256.0
    for c in range(C):
        _cdf_into(priv, c * NBINS, cdfb, csbuf)
        _cdf_into(priv, C * NBINS + c * NBINS, cdfb, ctbuf)

        _mid_first(ctbuf, mtbuf, 0, fbuf)

        def ibody(q, _):
            xs = csbuf[pl.ds(q * LANES, LANES)]
            ct255 = plsc.load_gather(ctbuf, [lane * 0 + (NBINS - 1)])
            r = _search(mtbuf, lane * 0, xs)
            ind1 = plsc.load_gather(fbuf, [r])
            ind0 = ind1 - 1
            neg = ind0 < 0
            i0_256 = jnp.where(neg, ind0 + NBINS, ind0)
            dx0 = plsc.load_gather(ctbuf, [i0_256])
            dx1 = plsc.load_gather(ctbuf, [ind1])
            dy0 = jnp.where(neg, ind0 + 2 * NBINS,
                            ind0).astype(jnp.float32) * inv256 - 1.0
            dy1 = ind1.astype(jnp.float32) * inv256 - 1.0
            interp = dy0 + (dy1 - dy0) * (xs - dx0) / (dx1 - dx0)
            pm = jnp.where(xs <= -1.0, -1.0,
                           jnp.where(xs >= ct255, G511, interp))
            pmbuf[pl.ds(q * LANES, LANES)] = pm
            return _

        lax.fori_loop(0, VB, ibody, None)

        _mid_first(csbuf, tbuf, c * NBINS, fbuf)

        def abody(k, _):
            rvec = lane + k * LANES
            i1 = plsc.load_gather(fbuf, [rvec])
            i0 = i1 - 1
            i0w = jnp.where(i0 < 0, i0 + NBINS, i0)
            dx0 = plsc.load_gather(csbuf, [i0w])
            dx1 = plsc.load_gather(csbuf, [i1])
            dy0 = plsc.load_gather(pmbuf, [i0w])
            dy1 = plsc.load_gather(pmbuf, [i1])
            b = (dy1 - dy0) / (dx1 - dx0)
            a = dy0 - b * dx0
            tbuf[pl.ds(TCOLS + c * NBINS + k * LANES, LANES)] = a
            tbuf[pl.ds(2 * TCOLS + c * NBINS + k * LANES, LANES)] = b
            return _

        lax.fori_loop(0, VB, abody, None)

        def clbody(k, _):
            cs255 = plsc.load_gather(csbuf, [lane * 0 + (NBINS - 1)])
            pm255 = plsc.load_gather(pmbuf, [lane * 0 + (NBINS - 1)])
            tbuf[pl.ds(3 * TCOLS + c * NBINS + k * LANES, LANES)] = cs255
            tbuf[pl.ds(4 * TCOLS + c * NBINS + k * LANES, LANES)] = pm255
            return _

        lax.fori_loop(0, VB, clbody, None)

    for c in range(C):
        pltpu.sync_copy(src_hbm.at[pl.ds(c * NPIX + wid * MSL, MSL)],
                        xbuf.at[pl.ds(c * MSL, MSL)])

    for c in range(C):
        cb = c * NBINS

        def mbody(g, _, c=c, cb=cb):
            s0 = c * MSL + g * (MU * LANES)
            xs = [xbuf[pl.ds(s0 + u * LANES, LANES)] for u in range(MU)]
            rs = [zeros16 + cb for _u in range(MU)]
            for step in (128, 64, 32, 16, 8, 4, 2, 1):
                for u in range(MU):
                    probe = rs[u] + step
                    mv = plsc.load_gather(tbuf, [probe])
                    rs[u] = jnp.where(mv < xs[u], probe, rs[u])
            ys = []
            for u in range(MU):
                a = plsc.load_gather(tbuf, [rs[u] + TCOLS])
                b = plsc.load_gather(tbuf, [rs[u] + 2 * TCOLS])
                thi = plsc.load_gather(tbuf, [rs[u] + 3 * TCOLS])
                vhi = plsc.load_gather(tbuf, [rs[u] + 4 * TCOLS])
                y = a + b * xs[u]
                y = jnp.where(xs[u] >= thi, vhi, y)
                ys.append(jnp.where(xs[u] <= -1.0, -1.0, y))
            for u in range(MU):
                xbuf[pl.ds(s0 + u * LANES, LANES)] = ys[u]
            return _

        lax.fori_loop(0, MSL // (MU * LANES), mbody, None)

    for c in range(C):
        pltpu.sync_copy(xbuf.at[pl.ds(c * MSL, MSL)],
                        out_hbm.at[pl.ds(c * NPIX + wid * MSL, MSL)])


def _sc_fused(src_f, tgt_f):
    mesh = plsc.VectorSubcoreMesh(
        core_axis_name="c", subcore_axis_name="s", num_cores=NC,
        num_subcores=NS)
    return pl.kernel(
        _fused_body,
        out_type=jax.ShapeDtypeStruct((NTOT,), jnp.float32),
        mesh=mesh,
        compiler_params=pltpu.CompilerParams(needs_layout_passes=False),
        scratch_types=[
            pltpu.VMEM((C * MSL,), jnp.float32),
            pltpu.VMEM((LANES * NHIST,), jnp.int32),
            pltpu.VMEM((NHIST,), jnp.int32),
            pltpu.VMEM_SHARED((NS * NHIST,), jnp.int32),
            pltpu.VMEM((5 * TCOLS,), jnp.float32),
            pltpu.VMEM((NBINS,), jnp.int32),
            pltpu.VMEM((NBINS,), jnp.float32),
            pltpu.VMEM((NBINS,), jnp.float32),
            pltpu.VMEM((NBINS,), jnp.float32),
            pltpu.VMEM((NBINS,), jnp.float32),
            pltpu.VMEM((NBINS,), jnp.int32),
        ],
    )(src_f, tgt_f)


def kernel(src, tgt):
    out_f = _sc_fused(_fwd(src), _fwd(tgt))
    return _bwd(out_f)


# speedup vs baseline: 3.3776x; 1.0612x over previous
import jax
import jax.numpy as jnp
from jax import lax
from jax.experimental import pallas as pl
from jax.experimental.pallas import tpu as pltpu
from jax.experimental.pallas import tpu_sc as plsc

NBINS = 256
H = 512
W = 512
C = 3
NPIX = H * W
NTOT = H * W * C
NC = 2
NS = 16
NW = NC * NS
LANES = 16
HSL = NPIX // NS
MSL = NPIX // NW
NHIST = 2 * C * NBINS
TCOLS = C * NBINS
SENTINEL = -3.0e38
G511 = 0.99609375
VB = NBINS // LANES
MU = 4


def _fwd(s):
    t = s.transpose(2, 0, 1).reshape(C, 64, 8, 4, 128)
    return t.transpose(0, 1, 3, 2, 4).reshape(-1)


def _bwd(f):
    t = f.reshape(C, 64, 4, 8, 128).transpose(0, 1, 3, 2, 4)
    return t.reshape(C, H, W).transpose(1, 2, 0)


def _lane_iota():
    return lax.broadcasted_iota(jnp.int32, (LANES,), 0)


def _bin_index(v):
    vc = jnp.minimum(jnp.maximum(v, -1.0), 1.0)
    t = (vc + 1.0) * 128.0
    return jnp.minimum(t.astype(jnp.int32), NBINS - 1)


def _cdf_into(hbuf, off, cdfb, csbuf):
    lane = _lane_iota()
    zeros16 = lane * 0

    def cbody(k, carry):
        v = hbuf[pl.ds(off + k * LANES, LANES)]
        s = plsc.cumsum(v) + carry
        cdfb[pl.ds(k * LANES, LANES)] = s
        return zeros16 + jnp.max(s)

    lax.fori_loop(0, VB, cbody, zeros16)
    cdfmin = jnp.min(cdfb[pl.ds(0, LANES)])

    def sbody(k, _):
        d = cdfb[pl.ds(k * LANES, LANES)] - cdfmin
        csbuf[pl.ds(k * LANES, LANES)] = (
            d.astype(jnp.float32) * 2.0 / float(NPIX - 1) - 1.0)
        return _

    lax.fori_loop(0, VB, sbody, None)


def _mid_first(tab, mbuf, moff, fbuf):
    lane = _lane_iota()
    zeros16 = lane * 0

    def body(k, fcarry):
        jvec = lane + k * LANES
        cur = plsc.load_gather(tab, [jvec])
        prev = plsc.load_gather(tab, [jnp.maximum(jvec - 1, 0)])
        m = (prev + cur) * 0.5
        m = jnp.where(jvec == 0, SENTINEL, m)
        mbuf[pl.ds(moff + k * LANES, LANES)] = m
        cand = jnp.where(cur != prev, jvec, 0)
        f = jnp.maximum(plsc.cummax(cand), fcarry)
        fbuf[pl.ds(k * LANES, LANES)] = f
        return zeros16 + jnp.max(f)

    lax.fori_loop(0, VB, body, zeros16)


def _search(mbuf, base, x):
    r = base
    for step in (128, 64, 32, 16, 8, 4, 2, 1):
        probe = r + step
        mv = plsc.load_gather(mbuf, [probe])
        r = jnp.where(mv < x, probe, r)
    return r


def _fused_body(src_hbm, tgt_hbm, out_hbm, xbuf, priv, red, shared, tbuf,
                cdfb, csbuf, ctbuf, mtbuf, pmbuf, fbuf,
                sh0, sh1, si0, si1, si2, so0, so1, so2):
    sid = lax.axis_index("s")
    cid = lax.axis_index("c")
    wid = sid * NC + cid

    lane = _lane_iota()
    zeros16 = lane * 0
    ones16 = zeros16 + 1

    hsems = (sh0, sh1)
    imgs = (src_hbm, tgt_hbm)

    def hslice(i):
        return imgs[i // C].at[pl.ds((i % C) * NPIX + sid * HSL, HSL)]

    handles = [None] * 6
    handles[0] = pltpu.make_async_copy(
        hslice(0), xbuf.at[pl.ds(0, HSL)], hsems[0])
    handles[0].start()

    def zero_body(i, _):
        priv[pl.ds(i * LANES, LANES)] = zeros16
        return _

    lax.fori_loop(0, (LANES * NHIST) // LANES, zero_body, None)

    HU = 4
    for i in range(6):
        half = (i % 2) * HSL
        if i < 5:
            nhalf = ((i + 1) % 2) * HSL
            handles[i + 1] = pltpu.make_async_copy(
                hslice(i + 1), xbuf.at[pl.ds(nhalf, HSL)],
                hsems[(i + 1) % 2])
            handles[i + 1].start()
        handles[i].wait()
        off = lane * NHIST + i * NBINS

        def hbody(g, _, off=off, half=half):
            for u in range(HU):
                v = xbuf[pl.ds(half + (g * HU + u) * LANES, LANES)]
                idx = _bin_index(v) + off
                plsc.addupdate_scatter(priv, [idx], ones16)
            return _

        lax.fori_loop(0, HSL // (HU * LANES), hbody, None)

    msems_in = (si0, si1, si2)
    min_handles = []
    for c in range(C):
        hdl = pltpu.make_async_copy(
            src_hbm.at[pl.ds(c * NPIX + wid * MSL, MSL)],
            xbuf.at[pl.ds(c * MSL, MSL)], msems_in[c])
        hdl.start()
        min_handles.append(hdl)

    def rbody(k, _):
        acc = priv[pl.ds(k * LANES, LANES)]
        for l in range(1, LANES):
            acc = acc + priv[pl.ds(l * NHIST + k * LANES, LANES)]
        red[pl.ds(k * LANES, LANES)] = acc
        return _

    lax.fori_loop(0, NHIST // LANES, rbody, None)

    pltpu.sync_copy(red, shared.at[pl.ds(sid * NHIST, NHIST)])
    plsc.subcore_barrier()
    pltpu.sync_copy(shared, priv)

    def r2body(k, _):
        acc = priv[pl.ds(k * LANES, LANES)]
        for t in range(1, NS):
            acc = acc + priv[pl.ds(t * NHIST + k * LANES, LANES)]
        priv[pl.ds(k * LANES, LANES)] = acc
        return _

    lax.fori_loop(0, NHIST // LANES, r2body, None)

    inv256 = 1.0 / 256.0
    for c in range(C):
        _cdf_into(priv, c * NBINS, cdfb, csbuf)
        _cdf_into(priv, C * NBINS + c * NBINS, cdfb, ctbuf)

        _mid_first(ctbuf, mtbuf, 0, fbuf)

        def ibody(q, _):
            xs = csbuf[pl.ds(q * LANES, LANES)]
            ct255 = plsc.load_gather(ctbuf, [lane * 0 + (NBINS - 1)])
            r = _search(mtbuf, lane * 0, xs)
            ind1 = plsc.load_gather(fbuf, [r])
            ind0 = ind1 - 1
            neg = ind0 < 0
            i0_256 = jnp.where(neg, ind0 + NBINS, ind0)
            dx0 = plsc.load_gather(ctbuf, [i0_256])
            dx1 = plsc.load_gather(ctbuf, [ind1])
            dy0 = jnp.where(neg, ind0 + 2 * NBINS,
                            ind0).astype(jnp.float32) * inv256 - 1.0
            dy1 = ind1.astype(jnp.float32) * inv256 - 1.0
            interp = dy0 + (dy1 - dy0) * (xs - dx0) / (dx1 - dx0)
            pm = jnp.where(xs <= -1.0, -1.0,
                           jnp.where(xs >= ct255, G511, interp))
            pmbuf[pl.ds(q * LANES, LANES)] = pm
            return _

        lax.fori_loop(0, VB, ibody, None)

        _mid_first(csbuf, tbuf, c * NBINS, fbuf)

        def abody(k, _):
            rvec = lane + k * LANES
            i1 = plsc.load_gather(fbuf, [rvec])
            i0 = i1 - 1
            i0w = jnp.where(i0 < 0, i0 + NBINS, i0)
            dx0 = plsc.load_gather(csbuf, [i0w])
            dx1 = plsc.load_gather(csbuf, [i1])
            dy0 = plsc.load_gather(pmbuf, [i0w])
            dy1 = plsc.load_gather(pmbuf, [i1])
            b = (dy1 - dy0) / (dx1 - dx0)
            a = dy0 - b * dx0
            tbuf[pl.ds(TCOLS + c * NBINS + k * LANES, LANES)] = a
            tbuf[pl.ds(2 * TCOLS + c * NBINS + k * LANES, LANES)] = b
            return _

        lax.fori_loop(0, VB, abody, None)

        def clbody(k, _):
            cs255 = plsc.load_gather(csbuf, [lane * 0 + (NBINS - 1)])
            pm255 = plsc.load_gather(pmbuf, [lane * 0 + (NBINS - 1)])
            tbuf[pl.ds(3 * TCOLS + c * NBINS + k * LANES, LANES)] = cs255
            tbuf[pl.ds(4 * TCOLS + c * NBINS + k * LANES, LANES)] = pm255
            return _

        lax.fori_loop(0, VB, clbody, None)

    msems_out = (so0, so1, so2)
    mout_handles = []
    for c in range(C):
        cb = c * NBINS
        min_handles[c].wait()

        def mbody(g, _, c=c, cb=cb):
            s0 = c * MSL + g * (MU * LANES)
            xs = [xbuf[pl.ds(s0 + u * LANES, LANES)] for u in range(MU)]
            rs = [zeros16 + cb for _u in range(MU)]
            for step in (128, 64, 32, 16, 8, 4, 2, 1):
                for u in range(MU):
                    probe = rs[u] + step
                    mv = plsc.load_gather(tbuf, [probe])
                    rs[u] = jnp.where(mv < xs[u], probe, rs[u])
            ys = []
            for u in range(MU):
                a = plsc.load_gather(tbuf, [rs[u] + TCOLS])
                b = plsc.load_gather(tbuf, [rs[u] + 2 * TCOLS])
                thi = plsc.load_gather(tbuf, [rs[u] + 3 * TCOLS])
                vhi = plsc.load_gather(tbuf, [rs[u] + 4 * TCOLS])
                y = a + b * xs[u]
                y = jnp.where(xs[u] >= thi, vhi, y)
                ys.append(jnp.where(xs[u] <= -1.0, -1.0, y))
            for u in range(MU):
                xbuf[pl.ds(s0 + u * LANES, LANES)] = ys[u]
            return _

        lax.fori_loop(0, MSL // (MU * LANES), mbody, None)
        hdl = pltpu.make_async_copy(
            xbuf.at[pl.ds(c * MSL, MSL)],
            out_hbm.at[pl.ds(c * NPIX + wid * MSL, MSL)], msems_out[c])
        hdl.start()
        mout_handles.append(hdl)

    for hdl in mout_handles:
        hdl.wait()


def _sc_fused(src_f, tgt_f):
    mesh = plsc.VectorSubcoreMesh(
        core_axis_name="c", subcore_axis_name="s", num_cores=NC,
        num_subcores=NS)
    return pl.kernel(
        _fused_body,
        out_type=jax.ShapeDtypeStruct((NTOT,), jnp.float32),
        mesh=mesh,
        compiler_params=pltpu.CompilerParams(needs_layout_passes=False),
        scratch_types=[
            pltpu.VMEM((2 * HSL,), jnp.float32),
            pltpu.VMEM((LANES * NHIST,), jnp.int32),
            pltpu.VMEM((NHIST,), jnp.int32),
            pltpu.VMEM_SHARED((NS * NHIST,), jnp.int32),
            pltpu.VMEM((5 * TCOLS,), jnp.float32),
            pltpu.VMEM((NBINS,), jnp.int32),
            pltpu.VMEM((NBINS,), jnp.float32),
            pltpu.VMEM((NBINS,), jnp.float32),
            pltpu.VMEM((NBINS,), jnp.float32),
            pltpu.VMEM((NBINS,), jnp.float32),
            pltpu.VMEM((NBINS,), jnp.int32),
            pltpu.SemaphoreType.DMA,
            pltpu.SemaphoreType.DMA,
            pltpu.SemaphoreType.DMA,
            pltpu.SemaphoreType.DMA,
            pltpu.SemaphoreType.DMA,
            pltpu.SemaphoreType.DMA,
            pltpu.SemaphoreType.DMA,
            pltpu.SemaphoreType.DMA,
        ],
    )(src_f, tgt_f)


def kernel(src, tgt):
    out_f = _sc_fused(_fwd(src), _fwd(tgt))
    return _bwd(out_f)


# speedup vs baseline: 4.0588x; 1.2017x over previous
import jax
import jax.numpy as jnp
from jax import lax
from jax.experimental import pallas as pl
from jax.experimental.pallas import tpu as pltpu
from jax.experimental.pallas import tpu_sc as plsc

NBINS = 256
H = 512
W = 512
C = 3
NPIX = H * W
NTOT = H * W * C
NC = 2
NS = 16
NW = NC * NS
LANES = 16
HSL = NPIX // NS
MSL = NPIX // NW
NHIST = 2 * C * NBINS
TCOLS = C * NBINS
SENTINEL = -3.0e38
G511 = 0.99609375
VB = NBINS // LANES
MU = 4


def _fwd(s):
    t = s.transpose(2, 0, 1).reshape(C, 64, 8, 4, 128)
    return t.transpose(0, 1, 3, 2, 4).reshape(-1)


def _bwd(f):
    t = f.reshape(C, 64, 4, 8, 128).transpose(0, 1, 3, 2, 4)
    return t.reshape(C, H, W).transpose(1, 2, 0)


def _lane_iota():
    return lax.broadcasted_iota(jnp.int32, (LANES,), 0)


def _bin_index(v):
    vc = jnp.minimum(jnp.maximum(v, -1.0), 1.0)
    t = (vc + 1.0) * 128.0
    return jnp.minimum(t.astype(jnp.int32), NBINS - 1)


def _cdf_into(hbuf, off, cdfb, csbuf):
    lane = _lane_iota()
    zeros16 = lane * 0

    def cbody(k, carry):
        v = hbuf[pl.ds(off + k * LANES, LANES)]
        s = plsc.cumsum(v) + carry
        cdfb[pl.ds(k * LANES, LANES)] = s
        return zeros16 + jnp.max(s)

    lax.fori_loop(0, VB, cbody, zeros16)
    cdfmin = jnp.min(cdfb[pl.ds(0, LANES)])

    def sbody(k, _):
        d = cdfb[pl.ds(k * LANES, LANES)] - cdfmin
        csbuf[pl.ds(k * LANES, LANES)] = (
            d.astype(jnp.float32) * 2.0 / float(NPIX - 1) - 1.0)
        return _

    lax.fori_loop(0, VB, sbody, None)


def _mid_first(tab, mbuf, moff, fbuf):
    lane = _lane_iota()
    zeros16 = lane * 0

    def body(k, fcarry):
        jvec = lane + k * LANES
        cur = plsc.load_gather(tab, [jvec])
        prev = plsc.load_gather(tab, [jnp.maximum(jvec - 1, 0)])
        m = (prev + cur) * 0.5
        m = jnp.where(jvec == 0, SENTINEL, m)
        mbuf[pl.ds(moff + k * LANES, LANES)] = m
        cand = jnp.where(cur != prev, jvec, 0)
        f = jnp.maximum(plsc.cummax(cand), fcarry)
        fbuf[pl.ds(k * LANES, LANES)] = f
        return zeros16 + jnp.max(f)

    lax.fori_loop(0, VB, body, zeros16)


def _search(mbuf, base, x):
    r = base
    for step in (128, 64, 32, 16, 8, 4, 2, 1):
        probe = r + step
        mv = plsc.load_gather(mbuf, [probe])
        r = jnp.where(mv < x, probe, r)
    return r


def _hist_body(src_hbm, tgt_hbm, out_hbm, xbuf, priv, red, sh0, sh1):
    sid = lax.axis_index("s")
    cid = lax.axis_index("c")
    wid = sid * NC + cid

    lane = _lane_iota()
    zeros16 = lane * 0
    ones16 = zeros16 + 1

    hsems = (sh0, sh1)
    imgs = (src_hbm, tgt_hbm)

    def hslice(i):
        return imgs[i // C].at[pl.ds((i % C) * NPIX + wid * MSL, MSL)]

    handles = [None] * 6
    handles[0] = pltpu.make_async_copy(
        hslice(0), xbuf.at[pl.ds(0, MSL)], hsems[0])
    handles[0].start()

    def zero_body(i, _):
        priv[pl.ds(i * LANES, LANES)] = zeros16
        return _

    lax.fori_loop(0, (LANES * NHIST) // LANES, zero_body, None)

    HU = 4
    for i in range(6):
        half = (i % 2) * MSL
        if i < 5:
            nhalf = ((i + 1) % 2) * MSL
            handles[i + 1] = pltpu.make_async_copy(
                hslice(i + 1), xbuf.at[pl.ds(nhalf, MSL)],
                hsems[(i + 1) % 2])
            handles[i + 1].start()
        handles[i].wait()
        off = lane * NHIST + i * NBINS

        def hbody(g, _, off=off, half=half):
            for u in range(HU):
                v = xbuf[pl.ds(half + (g * HU + u) * LANES, LANES)]
                idx = _bin_index(v) + off
                plsc.addupdate_scatter(priv, [idx], ones16)
            return _

        lax.fori_loop(0, MSL // (HU * LANES), hbody, None)

    def rbody(k, _):
        acc = priv[pl.ds(k * LANES, LANES)]
        for l in range(1, LANES):
            acc = acc + priv[pl.ds(l * NHIST + k * LANES, LANES)]
        red[pl.ds(k * LANES, LANES)] = acc
        return _

    lax.fori_loop(0, NHIST // LANES, rbody, None)
    pltpu.sync_copy(red, out_hbm.at[pl.ds(wid * NHIST, NHIST)])


def _map_body(src_hbm, hparts_hbm, out_hbm, xbuf, priv, tbuf,
              cdfb, csbuf, ctbuf, mtbuf, pmbuf, fbuf,
              sp, si0, si1, si2, so0, so1, so2):
    sid = lax.axis_index("s")
    cid = lax.axis_index("c")
    wid = sid * NC + cid

    lane = _lane_iota()
    zeros16 = lane * 0

    msems_in = (si0, si1, si2)
    min_handles = []
    for c in range(C):
        hdl = pltpu.make_async_copy(
            src_hbm.at[pl.ds(c * NPIX + wid * MSL, MSL)],
            xbuf.at[pl.ds(c * MSL, MSL)], msems_in[c])
        hdl.start()
        min_handles.append(hdl)

    ph = pltpu.make_async_copy(hparts_hbm, priv, sp)
    ph.start()
    ph.wait()

    def r2body(k, _):
        acc = priv[pl.ds(k * LANES, LANES)]
        for t in range(1, NW):
            acc = acc + priv[pl.ds(t * NHIST + k * LANES, LANES)]
        priv[pl.ds(k * LANES, LANES)] = acc
        return _

    lax.fori_loop(0, NHIST // LANES, r2body, None)

    inv256 = 1.0 / 256.0
    for c in range(C):
        _cdf_into(priv, c * NBINS, cdfb, csbuf)
        _cdf_into(priv, C * NBINS + c * NBINS, cdfb, ctbuf)

        _mid_first(ctbuf, mtbuf, 0, fbuf)

        def ibody(q, _):
            xs = csbuf[pl.ds(q * LANES, LANES)]
            ct255 = plsc.load_gather(ctbuf, [lane * 0 + (NBINS - 1)])
            r = _search(mtbuf, lane * 0, xs)
            ind1 = plsc.load_gather(fbuf, [r])
            ind0 = ind1 - 1
            neg = ind0 < 0
            i0_256 = jnp.where(neg, ind0 + NBINS, ind0)
            dx0 = plsc.load_gather(ctbuf, [i0_256])
            dx1 = plsc.load_gather(ctbuf, [ind1])
            dy0 = jnp.where(neg, ind0 + 2 * NBINS,
                            ind0).astype(jnp.float32) * inv256 - 1.0
            dy1 = ind1.astype(jnp.float32) * inv256 - 1.0
            interp = dy0 + (dy1 - dy0) * (xs - dx0) / (dx1 - dx0)
            pm = jnp.where(xs <= -1.0, -1.0,
                           jnp.where(xs >= ct255, G511, interp))
            pmbuf[pl.ds(q * LANES, LANES)] = pm
            return _

        lax.fori_loop(0, VB, ibody, None)

        _mid_first(csbuf, tbuf, c * NBINS, fbuf)

        def abody(k, _):
            rvec = lane + k * LANES
            i1 = plsc.load_gather(fbuf, [rvec])
            i0 = i1 - 1
            i0w = jnp.where(i0 < 0, i0 + NBINS, i0)
            dx0 = plsc.load_gather(csbuf, [i0w])
            dx1 = plsc.load_gather(csbuf, [i1])
            dy0 = plsc.load_gather(pmbuf, [i0w])
            dy1 = plsc.load_gather(pmbuf, [i1])
            b = (dy1 - dy0) / (dx1 - dx0)
            a = dy0 - b * dx0
            tbuf[pl.ds(TCOLS + c * NBINS + k * LANES, LANES)] = a
            tbuf[pl.ds(2 * TCOLS + c * NBINS + k * LANES, LANES)] = b
            return _

        lax.fori_loop(0, VB, abody, None)

        def clbody(k, _):
            cs255 = plsc.load_gather(csbuf, [lane * 0 + (NBINS - 1)])
            pm255 = plsc.load_gather(pmbuf, [lane * 0 + (NBINS - 1)])
            tbuf[pl.ds(3 * TCOLS + c * NBINS + k * LANES, LANES)] = cs255
            tbuf[pl.ds(4 * TCOLS + c * NBINS + k * LANES, LANES)] = pm255
            return _

        lax.fori_loop(0, VB, clbody, None)

    msems_out = (so0, so1, so2)
    mout_handles = []
    for c in range(C):
        cb = c * NBINS
        min_handles[c].wait()

        def mbody(g, _, c=c, cb=cb):
            s0 = c * MSL + g * (MU * LANES)
            xs = [xbuf[pl.ds(s0 + u * LANES, LANES)] for u in range(MU)]
            rs = [zeros16 + cb for _u in range(MU)]
            for step in (128, 64, 32, 16, 8, 4, 2, 1):
                for u in range(MU):
                    probe = rs[u] + step
                    mv = plsc.load_gather(tbuf, [probe])
                    rs[u] = jnp.where(mv < xs[u], probe, rs[u])
            ys = []
            for u in range(MU):
                a = plsc.load_gather(tbuf, [rs[u] + TCOLS])
                b = plsc.load_gather(tbuf, [rs[u] + 2 * TCOLS])
                thi = plsc.load_gather(tbuf, [rs[u] + 3 * TCOLS])
                vhi = plsc.load_gather(tbuf, [rs[u] + 4 * TCOLS])
                y = a + b * xs[u]
                y = jnp.where(xs[u] >= thi, vhi, y)
                ys.append(jnp.where(xs[u] <= -1.0, -1.0, y))
            for u in range(MU):
                xbuf[pl.ds(s0 + u * LANES, LANES)] = ys[u]
            return _

        lax.fori_loop(0, MSL // (MU * LANES), mbody, None)
        hdl = pltpu.make_async_copy(
            xbuf.at[pl.ds(c * MSL, MSL)],
            out_hbm.at[pl.ds(c * NPIX + wid * MSL, MSL)], msems_out[c])
        hdl.start()
        mout_handles.append(hdl)

    for hdl in mout_handles:
        hdl.wait()


def _mesh():
    return plsc.VectorSubcoreMesh(
        core_axis_name="c", subcore_axis_name="s", num_cores=NC,
        num_subcores=NS)


def _sc_hist(src_f, tgt_f):
    return pl.kernel(
        _hist_body,
        out_type=jax.ShapeDtypeStruct((NW * NHIST,), jnp.int32),
        mesh=_mesh(),
        compiler_params=pltpu.CompilerParams(needs_layout_passes=False),
        scratch_types=[
            pltpu.VMEM((2 * MSL,), jnp.float32),
            pltpu.VMEM((LANES * NHIST,), jnp.int32),
            pltpu.VMEM((NHIST,), jnp.int32),
            pltpu.SemaphoreType.DMA,
            pltpu.SemaphoreType.DMA,
        ],
    )(src_f, tgt_f)


def _sc_map(src_f, hparts):
    return pl.kernel(
        _map_body,
        out_type=jax.ShapeDtypeStruct((NTOT,), jnp.float32),
        mesh=_mesh(),
        compiler_params=pltpu.CompilerParams(needs_layout_passes=False),
        scratch_types=[
            pltpu.VMEM((C * MSL,), jnp.float32),
            pltpu.VMEM((NW * NHIST,), jnp.int32),
            pltpu.VMEM((5 * TCOLS,), jnp.float32),
            pltpu.VMEM((NBINS,), jnp.int32),
            pltpu.VMEM((NBINS,), jnp.float32),
            pltpu.VMEM((NBINS,), jnp.float32),
            pltpu.VMEM((NBINS,), jnp.float32),
            pltpu.VMEM((NBINS,), jnp.float32),
            pltpu.VMEM((NBINS,), jnp.int32),
            pltpu.SemaphoreType.DMA,
            pltpu.SemaphoreType.DMA,
            pltpu.SemaphoreType.DMA,
            pltpu.SemaphoreType.DMA,
            pltpu.SemaphoreType.DMA,
            pltpu.SemaphoreType.DMA,
            pltpu.SemaphoreType.DMA,
        ],
    )(src_f, hparts)


def kernel(src, tgt):
    src_f = _fwd(src)
    hparts = _sc_hist(src_f, _fwd(tgt))
    return _bwd(_sc_map(src_f, hparts))
